# baseline (speedup 1.0000x reference)
import functools

import jax
import jax.numpy as jnp
from jax import lax
from jax.experimental import pallas as pl
from jax.experimental.pallas import tpu as pltpu
from jax.experimental.pallas import tpu_sc as plsc

_NC = 2
_NS = 16
_L = 16
_NW = _NC * _NS
_NBUF = 3
_CHUNK = 8


@functools.cache
def _build(B, S, D, V):
    S_PER_W = S // _NW
    CHUNK = _CHUNK
    NCHUNK = S_PER_W // CHUNK
    ROWS = B * CHUNK
    DV = D // _L
    scale = jnp.float32(jnp.sqrt(jnp.float32(D)))

    mesh = plsc.VectorSubcoreMesh(
        core_axis_name="c", subcore_axis_name="s",
        num_cores=_NC, num_subcores=_NS,
    )

    @functools.partial(
        pl.kernel,
        out_type=jax.ShapeDtypeStruct((B, S, D), jnp.float32),
        mesh=mesh,
        scratch_types=[
            pltpu.VMEM((NCHUNK, ROWS), jnp.int32),
            pltpu.VMEM((_NBUF, ROWS, D), jnp.float32),
            pltpu.VMEM((_NBUF, CHUNK, D), jnp.float32),
            pltpu.SemaphoreType.DMA,
            pltpu.SemaphoreType.DMA,
            pltpu.SemaphoreType.DMA,
            pltpu.SemaphoreType.DMA,
            pltpu.SemaphoreType.DMA,
            pltpu.SemaphoreType.DMA,
        ],
    )
    def _k(xr_hbm, table_hbm, pos_hbm, out_hbm, idx_v, rows_v, pos_v,
           g0, g1, g2, o0, o1, o2):
        gsem = [g0, g1, g2]
        osem = [o0, o1, o2]
        w = lax.axis_index("s") * _NC + lax.axis_index("c")
        s_base = w * S_PER_W
        pltpu.sync_copy(xr_hbm.at[pl.ds(w * NCHUNK, NCHUNK)], idx_v)

        def start(c):
            nb = c % _NBUF
            half = ROWS // 2
            hs = [pltpu.async_copy(
                table_hbm.at[idx_v.at[c, pl.ds(i * half, half)]],
                rows_v.at[nb, pl.ds(i * half, half)], gsem[nb])
                for i in range(2)]
            hs.append(pltpu.async_copy(
                pos_hbm.at[pl.ds(s_base + c * CHUNK, CHUNK)],
                pos_v.at[nb], gsem[nb]))
            return hs

        inflight = {0: start(0)}
        if NCHUNK > 1:
            inflight[1] = start(1)
        outflight = {}

        for c in range(NCHUNK):
            nb = c % _NBUF
            for h in inflight.pop(c):
                h.wait()
            if c + 2 < NCHUNK:
                prev = c + 2 - _NBUF
                if prev >= 0:
                    for h in outflight.pop(prev):
                        h.wait()
                inflight[c + 2] = start(c + 2)

            def comp_d(d, carry):
                dsl = pl.ds(d * _L, _L)
                for j in range(CHUNK):
                    p = pos_v[nb, j, dsl]
                    for b in range(B):
                        r = b * CHUNK + j
                        rows_v[nb, r, dsl] = rows_v[nb, r, dsl] * scale + p
                return carry

            lax.fori_loop(0, DV, comp_d, 0, unroll=2)

            s0 = s_base + c * CHUNK
            hs = []
            for b in range(B):
                hs.append(pltpu.async_copy(
                    rows_v.at[nb, pl.ds(b * CHUNK, CHUNK)],
                    out_hbm.at[b, pl.ds(s0, CHUNK)], osem[nb]))
            outflight[c] = hs

        for hs in outflight.values():
            for h in hs:
                h.wait()

    return _k


def kernel(x, table, pos_encoding):
    B, S = x.shape
    V, D = table.shape
    _k = _build(B, S, D, V)
    S_PER_W = S // _NW
    NCHUNK = S_PER_W // _CHUNK
    xr = (
        x.astype(jnp.int32)
        .reshape(B, _NW, NCHUNK, _CHUNK)
        .transpose(1, 2, 0, 3)
        .reshape(_NW * NCHUNK, B * _CHUNK)
    )
    return _k(xr, table, pos_encoding[0, :S])


# speedup vs baseline: 2.6249x; 2.6249x over previous
import functools

import jax
import jax.numpy as jnp
from jax import lax
from jax.experimental import pallas as pl
from jax.experimental.pallas import tpu as pltpu
from jax.experimental.pallas import tpu_sc as plsc

_NC = 2
_NS = 16
_L = 16
_NW = _NC * _NS
_NBUF = 3
_CHUNK = 8


@functools.cache
def _build(B, S, D, V):
    S_PER_W = S // _NW
    CHUNK = _CHUNK
    NCHUNK = S_PER_W // CHUNK
    ROWS = B * CHUNK
    DV = D // _L
    scale = jnp.float32(jnp.sqrt(jnp.float32(D)))

    mesh = plsc.VectorSubcoreMesh(
        core_axis_name="c", subcore_axis_name="s",
        num_cores=_NC, num_subcores=_NS,
    )

    @functools.partial(
        pl.kernel,
        out_type=jax.ShapeDtypeStruct((B, S, D), jnp.float32),
        mesh=mesh,
        scratch_types=[
            pltpu.VMEM((NCHUNK, ROWS), jnp.int32),
            pltpu.VMEM((_NBUF, ROWS, D), jnp.float32),
            pltpu.VMEM((_NBUF, CHUNK, D), jnp.float32),
            pltpu.SemaphoreType.DMA,
            pltpu.SemaphoreType.DMA,
            pltpu.SemaphoreType.DMA,
            pltpu.SemaphoreType.DMA,
            pltpu.SemaphoreType.DMA,
            pltpu.SemaphoreType.DMA,
        ],
    )
    def _k(xr_hbm, table_hbm, pos_hbm, out_hbm, idx_v, rows_v, pos_v,
           g0, g1, g2, o0, o1, o2):
        gsem = [g0, g1, g2]
        osem = [o0, o1, o2]
        w = lax.axis_index("s") * _NC + lax.axis_index("c")
        s_base = w * S_PER_W
        pltpu.sync_copy(xr_hbm.at[pl.ds(w * NCHUNK, NCHUNK)], idx_v)

        def start(c):
            nb = c % _NBUF
            half = ROWS // 2
            hs = [pltpu.async_copy(
                table_hbm.at[idx_v.at[c, pl.ds(i * half, half)]],
                rows_v.at[nb, pl.ds(i * half, half)], gsem[nb])
                for i in range(2)]
            hs.append(pltpu.async_copy(
                pos_hbm.at[pl.ds(s_base + c * CHUNK, CHUNK)],
                pos_v.at[nb], gsem[nb]))
            return hs

        inflight = {0: start(0)}
        if NCHUNK > 1:
            inflight[1] = start(1)
        outflight = {}

        for c in range(NCHUNK):
            nb = c % _NBUF
            for h in inflight.pop(c):
                h.wait()
            if c + 2 < NCHUNK:
                prev = c + 2 - _NBUF
                if prev >= 0:
                    for h in outflight.pop(prev):
                        h.wait()
                inflight[c + 2] = start(c + 2)

            def comp_j(j, carry):
                def comp_d(d, carry2):
                    dsl = pl.ds(d * _L, _L)
                    p = pos_v[nb, j, dsl]
                    for b in range(B):
                        r = b * CHUNK + j
                        rows_v[nb, r, dsl] = rows_v[nb, r, dsl] * scale + p
                    return carry2
                return lax.fori_loop(0, DV, comp_d, carry, unroll=8)

            lax.fori_loop(0, CHUNK, comp_j, 0)

            s0 = s_base + c * CHUNK
            hs = []
            for b in range(B):
                hs.append(pltpu.async_copy(
                    rows_v.at[nb, pl.ds(b * CHUNK, CHUNK)],
                    out_hbm.at[b, pl.ds(s0, CHUNK)], osem[nb]))
            outflight[c] = hs

        for hs in outflight.values():
            for h in hs:
                h.wait()

    return _k


def kernel(x, table, pos_encoding):
    B, S = x.shape
    V, D = table.shape
    _k = _build(B, S, D, V)
    S_PER_W = S // _NW
    NCHUNK = S_PER_W // _CHUNK
    xr = (
        x.astype(jnp.int32)
        .reshape(B, _NW, NCHUNK, _CHUNK)
        .transpose(1, 2, 0, 3)
        .reshape(_NW * NCHUNK, B * _CHUNK)
    )
    return _k(xr, table, pos_encoding[0, :S])
